# baseline (speedup 1.0000x reference)
import jax
import jax.numpy as jnp
from jax import lax
from jax.experimental import pallas as pl
from jax.experimental.pallas import tpu as pltpu
from jax.experimental.pallas import tpu_sc as plsc

_B, _N, _K = 4, 2048, 16
_CIN = 128
_CP = 256
_CW = 128
_TNQ = 256
_TN = 256
_GW = 128


def _knn_body(q_ref, p_ref, o_ref):
    q = q_ref[0]
    p = p_ref[0]
    q2 = jnp.sum(q * q, axis=1, keepdims=True)
    p2 = jnp.sum(p * p, axis=0, keepdims=True)
    qp = jnp.dot(q, p, preferred_element_type=jnp.float32)
    d = q2 + p2 - 2.0 * qp
    iota = lax.broadcasted_iota(jnp.int32, d.shape, 1)
    for t in range(_K):
        m = jnp.min(d, axis=1, keepdims=True)
        am = jnp.min(jnp.where(d == m, iota, _N), axis=1, keepdims=True)
        o_ref[0, :, t] = am[:, 0]
        d = jnp.where(iota == am, jnp.float32(jnp.inf), d)


def _knn(q8, p8t):
    return pl.pallas_call(
        _knn_body,
        grid=(_B, _N // _TNQ),
        in_specs=[
            pl.BlockSpec((1, _TNQ, 8), lambda b, i: (b, i, 0)),
            pl.BlockSpec((1, 8, _N), lambda b, i: (b, 0, 0)),
        ],
        out_specs=pl.BlockSpec((1, _TNQ, _K), lambda b, i: (b, i, 0)),
        out_shape=jax.ShapeDtypeStruct((_B, _N, _K), jnp.int32),
    )(q8, p8t)


def _sc_gather(table, flat_idx):
    num = flat_idx.shape[1]

    @pl.kernel(
        out_type=jax.ShapeDtypeStruct((num, _CW), jnp.int32),
        mesh=plsc.VectorSubcoreMesh(core_axis_name="c", subcore_axis_name="s"),
    )
    def gather_kernel(t_hbm, i_hbm, o_hbm):
        def body(i_vmem, o_vmem):
            pltpu.sync_copy(t_hbm.at[i_vmem.at[0]], o_vmem)

        pltpu.emit_pipeline(
            body,
            grid=(num // _GW,),
            in_specs=[pl.BlockSpec((1, _GW), index_map=lambda i: (0, i))],
            out_specs=[pl.BlockSpec((_GW, _CW), index_map=lambda i: (i, 0))],
            core_axis_name=("c", "s"),
            dimension_semantics=(pltpu.PARALLEL,),
        )(i_hbm, o_hbm)

    return gather_kernel(table, flat_idx)


def _mlp_body(g_ref, pt_ref,
              w1e_ref, w1o_ref, w1q_ref, b1_ref,
              w2_ref, b2_ref, w3_ref, b3_ref,
              wse_ref, wso_ref, wsq_ref, bs_ref,
              wo1_ref, bo1_ref, wo2_ref, bo2_ref, o_ref):
    bf16 = jnp.bfloat16
    f32 = jnp.float32
    g = g_ref[0]
    ge = lax.bitcast_convert_type(jnp.left_shift(g, 16), f32).astype(bf16)
    go = lax.bitcast_convert_type(
        jnp.bitwise_and(g, jnp.int32(-65536)), f32).astype(bf16)
    pt = pt_ref[0].astype(bf16)

    gef = ge.reshape(_TN * _K, _CW)
    gof = go.reshape(_TN * _K, _CW)

    h = (jnp.dot(gef, w1e_ref[...], preferred_element_type=f32)
         + jnp.dot(gof, w1o_ref[...], preferred_element_type=f32))
    h = h.reshape(_TN, _K, 128)
    h = h - jnp.dot(pt, w1q_ref[...], preferred_element_type=f32)[:, None, :]
    h = jnp.maximum(h + b1_ref[...], 0.0)
    h = h.reshape(_TN * _K, 128).astype(bf16)
    h = jnp.maximum(jnp.dot(h, w2_ref[...], preferred_element_type=f32)
                    + b2_ref[...], 0.0).astype(bf16)
    h = jnp.maximum(jnp.dot(h, w3_ref[...], preferred_element_type=f32)
                    + b3_ref[...], 0.0)
    h = h.astype(bf16).reshape(_TN, _K, 256)

    gem = jnp.max(ge, axis=1)
    gom = jnp.max(go, axis=1)
    sk = (jnp.dot(gem, wse_ref[...], preferred_element_type=f32)
          + jnp.dot(gom, wso_ref[...], preferred_element_type=f32)
          - jnp.dot(pt, wsq_ref[...], preferred_element_type=f32))
    sk = jnp.maximum(sk + bs_ref[...], 0.0)

    acc = jnp.dot(h[:, 0, :], wo1_ref[0], preferred_element_type=f32)
    for k in range(1, _K):
        acc = acc + jnp.dot(h[:, k, :], wo1_ref[k],
                            preferred_element_type=f32)
    out1 = (jnp.maximum(acc + bo1_ref[...], 0.0) + sk).astype(bf16)
    out = jnp.maximum(jnp.dot(out1, wo2_ref[...], preferred_element_type=f32)
                      + bo2_ref[...], 0.0)
    o_ref[0] = out


def _mlp(g4, p8, weights):
    full = lambda shape: pl.BlockSpec(shape, lambda b, i: tuple(0 for _ in shape))
    w_specs = [
        full((_CW, 128)), full((_CW, 128)), full((8, 128)), full((1, 128)),
        full((128, 128)), full((1, 128)), full((128, 256)), full((1, 256)),
        full((_CW, 256)), full((_CW, 256)), full((8, 256)), full((1, 256)),
        full((_K, 256, 256)), full((1, 256)), full((256, 256)), full((1, 256)),
    ]
    return pl.pallas_call(
        _mlp_body,
        grid=(_B, _N // _TN),
        in_specs=[
            pl.BlockSpec((1, _TN, _K, _CW), lambda b, i: (b, i, 0, 0)),
            pl.BlockSpec((1, _TN, 8), lambda b, i: (b, i, 0)),
        ] + w_specs,
        out_specs=pl.BlockSpec((1, _TN, 256), lambda b, i: (b, i, 0)),
        out_shape=jax.ShapeDtypeStruct((_B, _N, 256), jnp.float32),
    )(g4, p8, *weights)


def kernel(points, point_features, query_points, W_skip, b_skip,
           W1, b1, W2, b2, W3, b3, Wo1, bo1, Wo2, bo2):
    f32 = jnp.float32
    pad5 = lambda x: jnp.pad(x, ((0, 0), (0, 0), (0, 5)))
    p8 = pad5(jnp.transpose(points, (0, 2, 1)))
    q8 = pad5(jnp.transpose(query_points, (0, 2, 1)))
    p8t = jnp.transpose(p8, (0, 2, 1))
    ft = jnp.transpose(point_features, (0, 2, 1))

    idx = _knn(q8, p8t)

    table = jnp.concatenate(
        [ft, p8[:, :, 0:3], jnp.zeros((_B, _N, _CP - _CIN - 3), f32)],
        axis=2).astype(jnp.bfloat16)
    table = lax.bitcast_convert_type(
        table.reshape(_B * _N, _CW, 2), jnp.int32)
    flat_idx = (idx + (jnp.arange(_B, dtype=jnp.int32) * _N)[:, None, None])
    flat_idx = flat_idx.reshape(1, _B * _N * _K)
    g = _sc_gather(table, flat_idx)
    g4 = g.reshape(_B, _N, _K, _CW)

    pad_w = lambda w: jnp.pad(w, ((0, 5), (0, 0)))
    W1t, W2t, W3t = W1.T, W2.T, W3.T
    Wst, Wo2t = W_skip.T, Wo2.T

    def tab_weights(Wt):
        out = Wt.shape[1]
        wtab = jnp.concatenate([
            Wt[3:3 + _CIN, :],
            Wt[0:3, :] + Wt[131:134, :],
            jnp.zeros((_CP - _CIN - 3, out), f32)], axis=0)
        return wtab[0::2, :], wtab[1::2, :]

    w1e, w1o = tab_weights(W1t)
    w1q = pad_w(W1t[131:134, :])
    wse, wso = tab_weights(Wst)
    wsq = pad_w(Wst[131:134, :])
    wo1t = jnp.transpose(Wo1, (1, 2, 0))
    row = lambda b: b.reshape(1, -1)
    bf = lambda w: w.astype(jnp.bfloat16)
    weights = [bf(w1e), bf(w1o), bf(w1q), row(b1), bf(W2t), row(b2),
               bf(W3t), row(b3), bf(wse), bf(wso), bf(wsq), row(b_skip),
               bf(wo1t), row(bo1), bf(Wo2t), row(bo2)]

    h = _mlp(g4, p8, weights)
    return (points, jnp.transpose(h, (0, 2, 1)))


# speedup vs baseline: 1.1166x; 1.1166x over previous
import jax
import jax.numpy as jnp
from jax import lax
from jax.experimental import pallas as pl
from jax.experimental.pallas import tpu as pltpu
from jax.experimental.pallas import tpu_sc as plsc

_B, _N, _K = 4, 2048, 16
_CIN = 128
_CP = 256
_CW = 128
_TNQ = 256
_TN = 256
_GW = 128


def _knn_body(q_ref, p_ref, o_ref):
    q = q_ref[0]
    p = p_ref[0]
    q2 = jnp.sum(q * q, axis=1, keepdims=True)
    p2 = jnp.sum(p * p, axis=0, keepdims=True)
    qp = jnp.dot(q, p, preferred_element_type=jnp.float32)
    d = q2 + p2 - 2.0 * qp
    iota = lax.broadcasted_iota(jnp.int32, d.shape, 1)
    for t in range(_K):
        m = jnp.min(d, axis=1, keepdims=True)
        am = jnp.min(jnp.where(d == m, iota, _N), axis=1, keepdims=True)
        o_ref[0, :, t] = am[:, 0]
        d = jnp.where(iota == am, jnp.float32(jnp.inf), d)


def _knn(q8, p8t):
    return pl.pallas_call(
        _knn_body,
        grid=(_B, _N // _TNQ),
        in_specs=[
            pl.BlockSpec((1, _TNQ, 8), lambda b, i: (b, i, 0)),
            pl.BlockSpec((1, 8, _N), lambda b, i: (b, 0, 0)),
        ],
        out_specs=pl.BlockSpec((1, _TNQ, _K), lambda b, i: (b, i, 0)),
        out_shape=jax.ShapeDtypeStruct((_B, _N, _K), jnp.int32),
    )(q8, p8t)


def _sc_gather(table, flat_idx):
    num = flat_idx.shape[1]

    @pl.kernel(
        out_type=jax.ShapeDtypeStruct((num, _CW), jnp.int32),
        mesh=plsc.VectorSubcoreMesh(core_axis_name="c", subcore_axis_name="s"),
    )
    def gather_kernel(t_hbm, i_hbm, o_hbm):
        def body(i_vmem, o_vmem):
            pltpu.sync_copy(t_hbm.at[i_vmem.at[0]], o_vmem)

        pltpu.emit_pipeline(
            body,
            grid=(num // _GW,),
            in_specs=[pl.BlockSpec((1, _GW), index_map=lambda i: (0, i))],
            out_specs=[pl.BlockSpec((_GW, _CW), index_map=lambda i: (i, 0))],
            core_axis_name=("c", "s"),
            dimension_semantics=(pltpu.PARALLEL,),
        )(i_hbm, o_hbm)

    return gather_kernel(table, flat_idx)


def _mlp_body(g_ref, pt_ref,
              w1e_ref, w1o_ref, w1q_ref, b1_ref,
              w2_ref, b2_ref, w3_ref, b3_ref,
              wse_ref, wso_ref, wsq_ref, bs_ref,
              wo1_ref, bo1_ref, wo2_ref, bo2_ref, o_ref):
    bf16 = jnp.bfloat16
    f32 = jnp.float32
    g = g_ref[0]
    ge = lax.bitcast_convert_type(jnp.left_shift(g, 16), f32).astype(bf16)
    go = lax.bitcast_convert_type(
        jnp.bitwise_and(g, jnp.int32(-65536)), f32).astype(bf16)
    pt = pt_ref[0].astype(bf16)

    gef = ge.reshape(_K * _TN, _CW)
    gof = go.reshape(_K * _TN, _CW)

    h = (jnp.dot(gef, w1e_ref[...], preferred_element_type=f32)
         + jnp.dot(gof, w1o_ref[...], preferred_element_type=f32))
    h = h.reshape(_K, _TN, 128)
    h = h - jnp.dot(pt, w1q_ref[...], preferred_element_type=f32)[None, :, :]
    h = jnp.maximum(h + b1_ref[...], 0.0)
    h = h.reshape(_K * _TN, 128).astype(bf16)
    h = jnp.maximum(jnp.dot(h, w2_ref[...], preferred_element_type=f32)
                    + b2_ref[...], 0.0).astype(bf16)
    h = jnp.maximum(jnp.dot(h, w3_ref[...], preferred_element_type=f32)
                    + b3_ref[...], 0.0)
    h = h.astype(bf16)

    gem = ge[0]
    gom = go[0]
    for k in range(1, _K):
        gem = jnp.maximum(gem, ge[k])
        gom = jnp.maximum(gom, go[k])
    sk = (jnp.dot(gem, wse_ref[...], preferred_element_type=f32)
          + jnp.dot(gom, wso_ref[...], preferred_element_type=f32)
          - jnp.dot(pt, wsq_ref[...], preferred_element_type=f32))
    sk = jnp.maximum(sk + bs_ref[...], 0.0)

    acc = jnp.dot(h[0:_TN], wo1_ref[0], preferred_element_type=f32)
    for k in range(1, _K):
        acc = acc + jnp.dot(h[k * _TN:(k + 1) * _TN], wo1_ref[k],
                            preferred_element_type=f32)
    out1 = (jnp.maximum(acc + bo1_ref[...], 0.0) + sk).astype(bf16)
    out = jnp.maximum(jnp.dot(out1, wo2_ref[...], preferred_element_type=f32)
                      + bo2_ref[...], 0.0)
    o_ref[0] = out


def _mlp(g4, p8, weights):
    full = lambda shape: pl.BlockSpec(shape, lambda b, i: tuple(0 for _ in shape))
    w_specs = [
        full((_CW, 128)), full((_CW, 128)), full((8, 128)), full((1, 128)),
        full((128, 128)), full((1, 128)), full((128, 256)), full((1, 256)),
        full((_CW, 256)), full((_CW, 256)), full((8, 256)), full((1, 256)),
        full((_K, 256, 256)), full((1, 256)), full((256, 256)), full((1, 256)),
    ]
    return pl.pallas_call(
        _mlp_body,
        grid=(_B, _N // _TN),
        in_specs=[
            pl.BlockSpec((1, _K, _TN, _CW), lambda b, i: (b, 0, i, 0)),
            pl.BlockSpec((1, _TN, 8), lambda b, i: (b, i, 0)),
        ] + w_specs,
        out_specs=pl.BlockSpec((1, _TN, 256), lambda b, i: (b, i, 0)),
        out_shape=jax.ShapeDtypeStruct((_B, _N, 256), jnp.float32),
    )(g4, p8, *weights)


def kernel(points, point_features, query_points, W_skip, b_skip,
           W1, b1, W2, b2, W3, b3, Wo1, bo1, Wo2, bo2):
    f32 = jnp.float32
    pad5 = lambda x: jnp.pad(x, ((0, 0), (0, 0), (0, 5)))
    p8 = pad5(jnp.transpose(points, (0, 2, 1)))
    q8 = pad5(jnp.transpose(query_points, (0, 2, 1)))
    p8t = jnp.transpose(p8, (0, 2, 1))
    ft = jnp.transpose(point_features, (0, 2, 1))

    idx = _knn(q8, p8t)

    table = jnp.concatenate(
        [ft, p8[:, :, 0:3], jnp.zeros((_B, _N, _CP - _CIN - 3), f32)],
        axis=2).astype(jnp.bfloat16)
    table = lax.bitcast_convert_type(
        table.reshape(_B * _N, _CW, 2), jnp.int32)
    idx_t = jnp.transpose(idx, (0, 2, 1))
    flat_idx = (idx_t + (jnp.arange(_B, dtype=jnp.int32) * _N)[:, None, None])
    flat_idx = flat_idx.reshape(1, _B * _N * _K)
    g = _sc_gather(table, flat_idx)
    g4 = g.reshape(_B, _K, _N, _CW)

    pad_w = lambda w: jnp.pad(w, ((0, 5), (0, 0)))
    W1t, W2t, W3t = W1.T, W2.T, W3.T
    Wst, Wo2t = W_skip.T, Wo2.T

    def tab_weights(Wt):
        out = Wt.shape[1]
        wtab = jnp.concatenate([
            Wt[3:3 + _CIN, :],
            Wt[0:3, :] + Wt[131:134, :],
            jnp.zeros((_CP - _CIN - 3, out), f32)], axis=0)
        return wtab[0::2, :], wtab[1::2, :]

    w1e, w1o = tab_weights(W1t)
    w1q = pad_w(W1t[131:134, :])
    wse, wso = tab_weights(Wst)
    wsq = pad_w(Wst[131:134, :])
    wo1t = jnp.transpose(Wo1, (1, 2, 0))
    row = lambda b: b.reshape(1, -1)
    bf = lambda w: w.astype(jnp.bfloat16)
    weights = [bf(w1e), bf(w1o), bf(w1q), row(b1), bf(W2t), row(b2),
               bf(W3t), row(b3), bf(wse), bf(wso), bf(wsq), row(b_skip),
               bf(wo1t), row(bo1), bf(Wo2t), row(bo2)]

    h = _mlp(g4, p8, weights)
    return (points, jnp.transpose(h, (0, 2, 1)))


# speedup vs baseline: 1.2476x; 1.1173x over previous
import jax
import jax.numpy as jnp
from jax import lax
from jax.experimental import pallas as pl
from jax.experimental.pallas import tpu as pltpu
from jax.experimental.pallas import tpu_sc as plsc

_B, _N, _K = 4, 2048, 16
_CIN = 128
_CP = 256
_CW = 128
_TNQ = 256
_TN = 256
_GW = 128


def _knn_body(q_ref, p_ref, o_ref):
    q = q_ref[0]
    p = p_ref[0]
    q2 = jnp.sum(q * q, axis=1, keepdims=True)
    p2 = jnp.sum(p * p, axis=0, keepdims=True)
    qp = jnp.dot(q, p, preferred_element_type=jnp.float32)
    d = q2 + p2 - 2.0 * qp
    iota = lax.broadcasted_iota(jnp.int32, d.shape, 1)
    bound = jnp.max(jnp.min(d.reshape(_TNQ, _K, _N // _K), axis=2),
                    axis=1, keepdims=True)
    scale = jnp.float32(2 ** 20 - 2) / jnp.maximum(bound, jnp.float32(1e-30))
    ki = jnp.clip((d * scale).astype(jnp.int32),
                  jnp.int32(-(2 ** 20)), jnp.int32(2 ** 20 - 1))
    key = jnp.bitwise_or(jnp.left_shift(ki, 11), iota)
    for t in range(_K):
        kmin = jnp.min(key, axis=1, keepdims=True)
        o_ref[0, :, t] = jnp.bitwise_and(kmin, jnp.int32(2047))[:, 0]
        key = jnp.where(key == kmin, jnp.int32(2 ** 31 - 1), key)


def _knn(q8, p8t):
    return pl.pallas_call(
        _knn_body,
        grid=(_B, _N // _TNQ),
        in_specs=[
            pl.BlockSpec((1, _TNQ, 8), lambda b, i: (b, i, 0)),
            pl.BlockSpec((1, 8, _N), lambda b, i: (b, 0, 0)),
        ],
        out_specs=pl.BlockSpec((1, _TNQ, _K), lambda b, i: (b, i, 0)),
        out_shape=jax.ShapeDtypeStruct((_B, _N, _K), jnp.int32),
    )(q8, p8t)


def _sc_gather(table, flat_idx):
    num = flat_idx.shape[1]

    @pl.kernel(
        out_type=jax.ShapeDtypeStruct((num, _CW), jnp.int32),
        mesh=plsc.VectorSubcoreMesh(core_axis_name="c", subcore_axis_name="s"),
    )
    def gather_kernel(t_hbm, i_hbm, o_hbm):
        def body(i_vmem, o_vmem):
            pltpu.sync_copy(t_hbm.at[i_vmem.at[0]], o_vmem)

        pltpu.emit_pipeline(
            body,
            grid=(num // _GW,),
            in_specs=[pl.BlockSpec((1, _GW), index_map=lambda i: (0, i))],
            out_specs=[pl.BlockSpec((_GW, _CW), index_map=lambda i: (i, 0))],
            core_axis_name=("c", "s"),
            dimension_semantics=(pltpu.PARALLEL,),
        )(i_hbm, o_hbm)

    return gather_kernel(table, flat_idx)


def _mlp_body(g_ref, pt_ref,
              w1e_ref, w1o_ref, w1q_ref, b1_ref,
              w2_ref, b2_ref, w3_ref, b3_ref,
              wse_ref, wso_ref, wsq_ref, bs_ref,
              wo1_ref, bo1_ref, wo2_ref, bo2_ref, o_ref):
    bf16 = jnp.bfloat16
    f32 = jnp.float32
    g = g_ref[0]
    ge = lax.bitcast_convert_type(jnp.left_shift(g, 16), f32).astype(bf16)
    go = lax.bitcast_convert_type(
        jnp.bitwise_and(g, jnp.int32(-65536)), f32).astype(bf16)
    pt = pt_ref[0].astype(bf16)

    gef = ge.reshape(_K * _TN, _CW)
    gof = go.reshape(_K * _TN, _CW)

    h = (jnp.dot(gef, w1e_ref[...], preferred_element_type=f32)
         + jnp.dot(gof, w1o_ref[...], preferred_element_type=f32))
    h = h.reshape(_K, _TN, 128)
    h = h - jnp.dot(pt, w1q_ref[...], preferred_element_type=f32)[None, :, :]
    h = jnp.maximum(h + b1_ref[...], 0.0)
    h = h.reshape(_K * _TN, 128).astype(bf16)
    h = jnp.maximum(jnp.dot(h, w2_ref[...], preferred_element_type=f32)
                    + b2_ref[...], 0.0).astype(bf16)
    h = jnp.maximum(jnp.dot(h, w3_ref[...], preferred_element_type=f32)
                    + b3_ref[...], 0.0)
    h = h.astype(bf16)

    gem = ge[0]
    gom = go[0]
    for k in range(1, _K):
        gem = jnp.maximum(gem, ge[k])
        gom = jnp.maximum(gom, go[k])
    sk = (jnp.dot(gem, wse_ref[...], preferred_element_type=f32)
          + jnp.dot(gom, wso_ref[...], preferred_element_type=f32)
          - jnp.dot(pt, wsq_ref[...], preferred_element_type=f32))
    sk = jnp.maximum(sk + bs_ref[...], 0.0)

    acc = jnp.dot(h[0:_TN], wo1_ref[0], preferred_element_type=f32)
    for k in range(1, _K):
        acc = acc + jnp.dot(h[k * _TN:(k + 1) * _TN], wo1_ref[k],
                            preferred_element_type=f32)
    out1 = (jnp.maximum(acc + bo1_ref[...], 0.0) + sk).astype(bf16)
    out = jnp.maximum(jnp.dot(out1, wo2_ref[...], preferred_element_type=f32)
                      + bo2_ref[...], 0.0)
    o_ref[0] = out


def _mlp(g4, p8, weights):
    full = lambda shape: pl.BlockSpec(shape, lambda b, i: tuple(0 for _ in shape))
    w_specs = [
        full((_CW, 128)), full((_CW, 128)), full((8, 128)), full((1, 128)),
        full((128, 128)), full((1, 128)), full((128, 256)), full((1, 256)),
        full((_CW, 256)), full((_CW, 256)), full((8, 256)), full((1, 256)),
        full((_K, 256, 256)), full((1, 256)), full((256, 256)), full((1, 256)),
    ]
    return pl.pallas_call(
        _mlp_body,
        grid=(_B, _N // _TN),
        in_specs=[
            pl.BlockSpec((1, _K, _TN, _CW), lambda b, i: (b, 0, i, 0)),
            pl.BlockSpec((1, _TN, 8), lambda b, i: (b, i, 0)),
        ] + w_specs,
        out_specs=pl.BlockSpec((1, _TN, 256), lambda b, i: (b, i, 0)),
        out_shape=jax.ShapeDtypeStruct((_B, _N, 256), jnp.float32),
    )(g4, p8, *weights)


def kernel(points, point_features, query_points, W_skip, b_skip,
           W1, b1, W2, b2, W3, b3, Wo1, bo1, Wo2, bo2):
    f32 = jnp.float32
    pad5 = lambda x: jnp.pad(x, ((0, 0), (0, 0), (0, 5)))
    p8 = pad5(jnp.transpose(points, (0, 2, 1)))
    q8 = pad5(jnp.transpose(query_points, (0, 2, 1)))
    p8t = jnp.transpose(p8, (0, 2, 1))
    ft = jnp.transpose(point_features, (0, 2, 1))

    idx = _knn(q8, p8t)

    table = jnp.concatenate(
        [ft, p8[:, :, 0:3], jnp.zeros((_B, _N, _CP - _CIN - 3), f32)],
        axis=2).astype(jnp.bfloat16)
    table = lax.bitcast_convert_type(
        table.reshape(_B * _N, _CW, 2), jnp.int32)
    idx_t = jnp.transpose(idx, (0, 2, 1))
    flat_idx = (idx_t + (jnp.arange(_B, dtype=jnp.int32) * _N)[:, None, None])
    flat_idx = flat_idx.reshape(1, _B * _N * _K)
    g = _sc_gather(table, flat_idx)
    g4 = g.reshape(_B, _K, _N, _CW)

    pad_w = lambda w: jnp.pad(w, ((0, 5), (0, 0)))
    W1t, W2t, W3t = W1.T, W2.T, W3.T
    Wst, Wo2t = W_skip.T, Wo2.T

    def tab_weights(Wt):
        out = Wt.shape[1]
        wtab = jnp.concatenate([
            Wt[3:3 + _CIN, :],
            Wt[0:3, :] + Wt[131:134, :],
            jnp.zeros((_CP - _CIN - 3, out), f32)], axis=0)
        return wtab[0::2, :], wtab[1::2, :]

    w1e, w1o = tab_weights(W1t)
    w1q = pad_w(W1t[131:134, :])
    wse, wso = tab_weights(Wst)
    wsq = pad_w(Wst[131:134, :])
    wo1t = jnp.transpose(Wo1, (1, 2, 0))
    row = lambda b: b.reshape(1, -1)
    bf = lambda w: w.astype(jnp.bfloat16)
    weights = [bf(w1e), bf(w1o), bf(w1q), row(b1), bf(W2t), row(b2),
               bf(W3t), row(b3), bf(wse), bf(wso), bf(wsq), row(b_skip),
               bf(wo1t), row(bo1), bf(Wo2t), row(bo2)]

    h = _mlp(g4, p8, weights)
    return (points, jnp.transpose(h, (0, 2, 1)))


# speedup vs baseline: 1.5023x; 1.2042x over previous
import jax
import jax.numpy as jnp
from jax import lax
from jax.experimental import pallas as pl
from jax.experimental.pallas import tpu as pltpu
from jax.experimental.pallas import tpu_sc as plsc

_B, _N, _K = 4, 2048, 16
_CIN = 128
_CP = 256
_CW = 128
_TNQ = 256
_TN = 256
_GW = 128


def _knn_body(q_ref, p_ref, o_ref):
    q = q_ref[0]
    p = p_ref[0]
    q2 = jnp.sum(q * q, axis=1, keepdims=True)
    p2 = jnp.sum(p * p, axis=0, keepdims=True)
    qp = jnp.dot(q, p, preferred_element_type=jnp.float32)
    d = q2 + p2 - 2.0 * qp
    iota = lax.broadcasted_iota(jnp.int32, d.shape, 1)
    bound = jnp.max(jnp.min(d.reshape(_TNQ, _K, _N // _K), axis=2),
                    axis=1, keepdims=True)
    scale = jnp.float32(2 ** 19 - 2) / jnp.maximum(bound, jnp.float32(1e-30))
    ki = jnp.clip((d * scale).astype(jnp.int32),
                  jnp.int32(0), jnp.int32(2 ** 19 - 1))
    key = lax.bitcast_convert_type(
        jnp.bitwise_or(jnp.left_shift(ki, 11), iota) + jnp.int32(2 ** 28),
        jnp.float32)
    big = jnp.float32(1e30)
    prev = jnp.zeros((_TNQ, 1), jnp.float32)
    cols = []
    for t in range(_K):
        prev = jnp.min(jnp.where(key > prev, key, big), axis=1, keepdims=True)
        cols.append(prev)
    ids = jnp.bitwise_and(
        lax.bitcast_convert_type(jnp.concatenate(cols, axis=1), jnp.int32),
        jnp.int32(2047))
    o_ref[0] = jnp.transpose(ids) + pl.program_id(0) * _N


def _knn(q8, p8t):
    return pl.pallas_call(
        _knn_body,
        grid=(_B, _N // _TNQ),
        in_specs=[
            pl.BlockSpec((1, _TNQ, 8), lambda b, i: (b, i, 0)),
            pl.BlockSpec((1, 8, _N), lambda b, i: (b, 0, 0)),
        ],
        out_specs=pl.BlockSpec((1, _K, _TNQ), lambda b, i: (b, 0, i)),
        out_shape=jax.ShapeDtypeStruct((_B, _K, _N), jnp.int32),
    )(q8, p8t)


def _sc_gather(table, flat_idx):
    num = flat_idx.shape[1]

    @pl.kernel(
        out_type=jax.ShapeDtypeStruct((num, _CW), jnp.int32),
        mesh=plsc.VectorSubcoreMesh(core_axis_name="c", subcore_axis_name="s"),
    )
    def gather_kernel(t_hbm, i_hbm, o_hbm):
        def body(i_vmem, o_vmem):
            pltpu.sync_copy(t_hbm.at[i_vmem.at[0]], o_vmem)

        pltpu.emit_pipeline(
            body,
            grid=(num // _GW,),
            in_specs=[pl.BlockSpec((1, _GW), index_map=lambda i: (0, i))],
            out_specs=[pl.BlockSpec((_GW, _CW), index_map=lambda i: (i, 0))],
            core_axis_name=("c", "s"),
            dimension_semantics=(pltpu.PARALLEL,),
        )(i_hbm, o_hbm)

    return gather_kernel(table, flat_idx)


def _mlp_body(g_ref, pt_ref,
              w1e_ref, w1o_ref, w1q_ref, b1_ref,
              w2_ref, b2_ref, w3_ref, b3_ref,
              wse_ref, wso_ref, wsq_ref, bs_ref,
              wo1_ref, bo1_ref, wo2_ref, bo2_ref, o_ref):
    bf16 = jnp.bfloat16
    f32 = jnp.float32
    g = g_ref[0]
    ge = lax.bitcast_convert_type(jnp.left_shift(g, 16), f32).astype(bf16)
    go = lax.bitcast_convert_type(
        jnp.bitwise_and(g, jnp.int32(-65536)), f32).astype(bf16)
    pt = pt_ref[0].astype(bf16)

    gef = ge.reshape(_K * _TN, _CW)
    gof = go.reshape(_K * _TN, _CW)

    h = (jnp.dot(gef, w1e_ref[...], preferred_element_type=f32)
         + jnp.dot(gof, w1o_ref[...], preferred_element_type=f32))
    h = h.reshape(_K, _TN, 128)
    h = h - jnp.dot(pt, w1q_ref[...], preferred_element_type=f32)[None, :, :]
    h = jnp.maximum(h + b1_ref[...], 0.0)
    h = h.reshape(_K * _TN, 128).astype(bf16)
    h = jnp.maximum(jnp.dot(h, w2_ref[...], preferred_element_type=f32)
                    + b2_ref[...], 0.0).astype(bf16)
    h = jnp.maximum(jnp.dot(h, w3_ref[...], preferred_element_type=f32)
                    + b3_ref[...], 0.0)
    h = h.astype(bf16)

    gem = ge[0]
    gom = go[0]
    for k in range(1, _K):
        gem = jnp.maximum(gem, ge[k])
        gom = jnp.maximum(gom, go[k])
    sk = (jnp.dot(gem, wse_ref[...], preferred_element_type=f32)
          + jnp.dot(gom, wso_ref[...], preferred_element_type=f32)
          - jnp.dot(pt, wsq_ref[...], preferred_element_type=f32))
    sk = jnp.maximum(sk + bs_ref[...], 0.0)

    acc = jnp.dot(h[0:_TN], wo1_ref[0], preferred_element_type=f32)
    for k in range(1, _K):
        acc = acc + jnp.dot(h[k * _TN:(k + 1) * _TN], wo1_ref[k],
                            preferred_element_type=f32)
    out1 = (jnp.maximum(acc + bo1_ref[...], 0.0) + sk).astype(bf16)
    out = jnp.maximum(jnp.dot(out1, wo2_ref[...], preferred_element_type=f32)
                      + bo2_ref[...], 0.0)
    o_ref[0] = out


def _mlp(g4, p8, weights):
    full = lambda shape: pl.BlockSpec(shape, lambda b, i: tuple(0 for _ in shape))
    w_specs = [
        full((_CW, 128)), full((_CW, 128)), full((8, 128)), full((1, 128)),
        full((128, 128)), full((1, 128)), full((128, 256)), full((1, 256)),
        full((_CW, 256)), full((_CW, 256)), full((8, 256)), full((1, 256)),
        full((_K, 256, 256)), full((1, 256)), full((256, 256)), full((1, 256)),
    ]
    return pl.pallas_call(
        _mlp_body,
        grid=(_B, _N // _TN),
        in_specs=[
            pl.BlockSpec((1, _K, _TN, _CW), lambda b, i: (b, 0, i, 0)),
            pl.BlockSpec((1, _TN, 8), lambda b, i: (b, i, 0)),
        ] + w_specs,
        out_specs=pl.BlockSpec((1, _TN, 256), lambda b, i: (b, i, 0)),
        out_shape=jax.ShapeDtypeStruct((_B, _N, 256), jnp.float32),
    )(g4, p8, *weights)


def kernel(points, point_features, query_points, W_skip, b_skip,
           W1, b1, W2, b2, W3, b3, Wo1, bo1, Wo2, bo2):
    f32 = jnp.float32
    pad5 = lambda x: jnp.pad(x, ((0, 0), (0, 0), (0, 5)))
    p8 = pad5(jnp.transpose(points, (0, 2, 1)))
    q8 = pad5(jnp.transpose(query_points, (0, 2, 1)))
    p8t = jnp.transpose(p8, (0, 2, 1))
    ft = jnp.transpose(point_features, (0, 2, 1))

    gidx = _knn(q8, p8t)

    table = jnp.concatenate(
        [ft, p8[:, :, 0:3], jnp.zeros((_B, _N, _CP - _CIN - 3), f32)],
        axis=2).astype(jnp.bfloat16)
    table = lax.bitcast_convert_type(
        table.reshape(_B * _N, _CW, 2), jnp.int32)
    flat_idx = gidx.reshape(1, _B * _N * _K)
    g = _sc_gather(table, flat_idx)
    g4 = g.reshape(_B, _K, _N, _CW)

    pad_w = lambda w: jnp.pad(w, ((0, 5), (0, 0)))
    W1t, W2t, W3t = W1.T, W2.T, W3.T
    Wst, Wo2t = W_skip.T, Wo2.T

    def tab_weights(Wt):
        out = Wt.shape[1]
        wtab = jnp.concatenate([
            Wt[3:3 + _CIN, :],
            Wt[0:3, :] + Wt[131:134, :],
            jnp.zeros((_CP - _CIN - 3, out), f32)], axis=0)
        return wtab[0::2, :], wtab[1::2, :]

    w1e, w1o = tab_weights(W1t)
    w1q = pad_w(W1t[131:134, :])
    wse, wso = tab_weights(Wst)
    wsq = pad_w(Wst[131:134, :])
    wo1t = jnp.transpose(Wo1, (1, 2, 0))
    row = lambda b: b.reshape(1, -1)
    bf = lambda w: w.astype(jnp.bfloat16)
    weights = [bf(w1e), bf(w1o), bf(w1q), row(b1), bf(W2t), row(b2),
               bf(W3t), row(b3), bf(wse), bf(wso), bf(wsq), row(b_skip),
               bf(wo1t), row(bo1), bf(Wo2t), row(bo2)]

    h = _mlp(g4, p8, weights)
    return (points, jnp.transpose(h, (0, 2, 1)))


# speedup vs baseline: 1.5103x; 1.0053x over previous
import jax
import jax.numpy as jnp
from jax import lax
from jax.experimental import pallas as pl
from jax.experimental.pallas import tpu as pltpu
from jax.experimental.pallas import tpu_sc as plsc

_B, _N, _K = 4, 2048, 16
_CIN = 128
_CP = 256
_CW = 128
_TNQ = 256
_TN = 256
_GW = 128


def _knn_body(q_ref, p_ref, o_ref):
    q = q_ref[0]
    p = p_ref[0]
    q2 = jnp.sum(q * q, axis=1, keepdims=True)
    p2 = jnp.sum(p * p, axis=0, keepdims=True)
    qp = jnp.dot(q, p, preferred_element_type=jnp.float32)
    d = q2 + p2 - 2.0 * qp
    iota = lax.broadcasted_iota(jnp.int32, d.shape, 1)
    bound = jnp.max(jnp.min(d.reshape(_TNQ, _K, _N // _K), axis=2),
                    axis=1, keepdims=True)
    scale = jnp.float32(2 ** 19 - 2) / jnp.maximum(bound, jnp.float32(1e-30))
    ki = jnp.clip((d * scale).astype(jnp.int32),
                  jnp.int32(0), jnp.int32(2 ** 19 - 1))
    key = lax.bitcast_convert_type(
        jnp.bitwise_or(jnp.left_shift(ki, 11), iota) + jnp.int32(2 ** 28),
        jnp.float32)
    big = jnp.float32(1e30)
    prev = jnp.zeros((_TNQ, 1), jnp.float32)
    cols = []
    for t in range(_K):
        prev = jnp.min(jnp.where(key > prev, key, big), axis=1, keepdims=True)
        cols.append(prev)
    ids = jnp.bitwise_and(
        lax.bitcast_convert_type(jnp.concatenate(cols, axis=1), jnp.int32),
        jnp.int32(2047))
    o_ref[0] = jnp.transpose(ids) + pl.program_id(0) * _N


def _knn(q8, p8t):
    return pl.pallas_call(
        _knn_body,
        grid=(_B, _N // _TNQ),
        in_specs=[
            pl.BlockSpec((1, _TNQ, 8), lambda b, i: (b, i, 0)),
            pl.BlockSpec((1, 8, _N), lambda b, i: (b, 0, 0)),
        ],
        out_specs=pl.BlockSpec((1, _K, _TNQ), lambda b, i: (b, 0, i)),
        out_shape=jax.ShapeDtypeStruct((_B, _K, _N), jnp.int32),
    )(q8, p8t)


def _sc_gather(table, idx2):
    rows, n = idx2.shape
    num = rows * n

    @pl.kernel(
        out_type=jax.ShapeDtypeStruct((num, _CW), jnp.int32),
        mesh=plsc.VectorSubcoreMesh(core_axis_name="c", subcore_axis_name="s"),
    )
    def gather_kernel(t_hbm, i_hbm, o_hbm):
        def body(i_vmem, o_vmem):
            pltpu.sync_copy(t_hbm.at[i_vmem.at[0]], o_vmem)

        pltpu.emit_pipeline(
            body,
            grid=(rows, n // _GW),
            in_specs=[pl.BlockSpec((1, _GW), index_map=lambda r, c: (r, c))],
            out_specs=[pl.BlockSpec((_GW, _CW),
                                    index_map=lambda r, c: (r * (n // _GW) + c, 0))],
            core_axis_name=("c", "s"),
            dimension_semantics=(pltpu.PARALLEL, pltpu.PARALLEL),
        )(i_hbm, o_hbm)

    return gather_kernel(table, idx2)


def _mlp_body(g_ref, pt_ref,
              w1e_ref, w1o_ref, w1q_ref, b1_ref,
              w2_ref, b2_ref, w3_ref, b3_ref,
              wse_ref, wso_ref, wsq_ref, bs_ref,
              wo1_ref, bo1_ref, wo2_ref, bo2_ref, o_ref):
    bf16 = jnp.bfloat16
    f32 = jnp.float32
    g = g_ref[0]
    ge = lax.bitcast_convert_type(jnp.left_shift(g, 16), f32).astype(bf16)
    go = lax.bitcast_convert_type(
        jnp.bitwise_and(g, jnp.int32(-65536)), f32).astype(bf16)
    pt = pt_ref[0].astype(bf16)

    gef = ge.reshape(_K * _TN, _CW)
    gof = go.reshape(_K * _TN, _CW)

    h = (jnp.dot(gef, w1e_ref[...], preferred_element_type=f32)
         + jnp.dot(gof, w1o_ref[...], preferred_element_type=f32))
    h = h.reshape(_K, _TN, 128)
    h = h - jnp.dot(pt, w1q_ref[...], preferred_element_type=f32)[None, :, :]
    h = jnp.maximum(h + b1_ref[...], 0.0)
    h = h.reshape(_K * _TN, 128).astype(bf16)
    h = jnp.maximum(jnp.dot(h, w2_ref[...], preferred_element_type=f32)
                    + b2_ref[...], 0.0).astype(bf16)
    h = jnp.maximum(jnp.dot(h, w3_ref[...], preferred_element_type=f32)
                    + b3_ref[...], 0.0)
    h = h.astype(bf16)

    gem = ge[0]
    gom = go[0]
    for k in range(1, _K):
        gem = jnp.maximum(gem, ge[k])
        gom = jnp.maximum(gom, go[k])
    sk = (jnp.dot(gem, wse_ref[...], preferred_element_type=f32)
          + jnp.dot(gom, wso_ref[...], preferred_element_type=f32)
          - jnp.dot(pt, wsq_ref[...], preferred_element_type=f32))
    sk = jnp.maximum(sk + bs_ref[...], 0.0)

    acc = jnp.dot(h[0:_TN], wo1_ref[0], preferred_element_type=f32)
    for k in range(1, _K):
        acc = acc + jnp.dot(h[k * _TN:(k + 1) * _TN], wo1_ref[k],
                            preferred_element_type=f32)
    out1 = (jnp.maximum(acc + bo1_ref[...], 0.0) + sk).astype(bf16)
    out = jnp.maximum(jnp.dot(out1, wo2_ref[...], preferred_element_type=f32)
                      + bo2_ref[...], 0.0)
    o_ref[0] = out


def _mlp(g4, p8, weights):
    full = lambda shape: pl.BlockSpec(shape, lambda b, i: tuple(0 for _ in shape))
    w_specs = [
        full((_CW, 128)), full((_CW, 128)), full((8, 128)), full((1, 128)),
        full((128, 128)), full((1, 128)), full((128, 256)), full((1, 256)),
        full((_CW, 256)), full((_CW, 256)), full((8, 256)), full((1, 256)),
        full((_K, 256, 256)), full((1, 256)), full((256, 256)), full((1, 256)),
    ]
    return pl.pallas_call(
        _mlp_body,
        grid=(_B, _N // _TN),
        in_specs=[
            pl.BlockSpec((1, _K, _TN, _CW), lambda b, i: (b, 0, i, 0)),
            pl.BlockSpec((1, _TN, 8), lambda b, i: (b, i, 0)),
        ] + w_specs,
        out_specs=pl.BlockSpec((1, _TN, 256), lambda b, i: (b, i, 0)),
        out_shape=jax.ShapeDtypeStruct((_B, _N, 256), jnp.float32),
    )(g4, p8, *weights)


def kernel(points, point_features, query_points, W_skip, b_skip,
           W1, b1, W2, b2, W3, b3, Wo1, bo1, Wo2, bo2):
    f32 = jnp.float32
    pad5 = lambda x: jnp.pad(x, ((0, 0), (0, 0), (0, 5)))
    p8 = pad5(jnp.transpose(points, (0, 2, 1)))
    q8 = pad5(jnp.transpose(query_points, (0, 2, 1)))
    p8t = jnp.transpose(p8, (0, 2, 1))
    ft = jnp.transpose(point_features, (0, 2, 1))

    gidx = _knn(q8, p8t)

    table = jnp.concatenate(
        [ft, p8[:, :, 0:3], jnp.zeros((_B, _N, _CP - _CIN - 3), f32)],
        axis=2).astype(jnp.bfloat16)
    table = lax.bitcast_convert_type(
        table.reshape(_B * _N, _CW, 2), jnp.int32)
    g = _sc_gather(table, gidx.reshape(_B * _K, _N))
    g4 = g.reshape(_B, _K, _N, _CW)

    pad_w = lambda w: jnp.pad(w, ((0, 5), (0, 0)))
    W1t, W2t, W3t = W1.T, W2.T, W3.T
    Wst, Wo2t = W_skip.T, Wo2.T

    def tab_weights(Wt):
        out = Wt.shape[1]
        wtab = jnp.concatenate([
            Wt[3:3 + _CIN, :],
            Wt[0:3, :] + Wt[131:134, :],
            jnp.zeros((_CP - _CIN - 3, out), f32)], axis=0)
        return wtab[0::2, :], wtab[1::2, :]

    w1e, w1o = tab_weights(W1t)
    w1q = pad_w(W1t[131:134, :])
    wse, wso = tab_weights(Wst)
    wsq = pad_w(Wst[131:134, :])
    wo1t = jnp.transpose(Wo1, (1, 2, 0))
    row = lambda b: b.reshape(1, -1)
    bf = lambda w: w.astype(jnp.bfloat16)
    weights = [bf(w1e), bf(w1o), bf(w1q), row(b1), bf(W2t), row(b2),
               bf(W3t), row(b3), bf(wse), bf(wso), bf(wsq), row(b_skip),
               bf(wo1t), row(bo1), bf(Wo2t), row(bo2)]

    h = _mlp(g4, p8, weights)
    return (points, jnp.transpose(h, (0, 2, 1)))


# speedup vs baseline: 1.5969x; 1.0573x over previous
import jax
import jax.numpy as jnp
from jax import lax
from jax.experimental import pallas as pl
from jax.experimental.pallas import tpu as pltpu
from jax.experimental.pallas import tpu_sc as plsc

_B, _N, _K = 4, 2048, 16
_CIN = 128
_CP = 256
_CW = 128
_TNQ = 256
_TN = 256
_GW = 128


def _knn_body(q_ref, p_ref, o_ref):
    q = q_ref[0]
    p = p_ref[0]
    q2 = jnp.sum(q * q, axis=1, keepdims=True)
    p2 = jnp.sum(p * p, axis=0, keepdims=True)
    qp = jnp.dot(q, p, preferred_element_type=jnp.float32)
    d = q2 + p2 - 2.0 * qp
    iota = lax.broadcasted_iota(jnp.int32, d.shape, 1)
    bound = jnp.max(jnp.min(d.reshape(_TNQ, _K, _N // _K), axis=2),
                    axis=1, keepdims=True)
    scale = jnp.float32(2 ** 19 - 2) / jnp.maximum(bound, jnp.float32(1e-30))
    ki = jnp.clip((d * scale).astype(jnp.int32),
                  jnp.int32(0), jnp.int32(2 ** 19 - 1))
    key = lax.bitcast_convert_type(
        jnp.bitwise_or(jnp.left_shift(ki, 11), iota) + jnp.int32(2 ** 28),
        jnp.float32)
    big = jnp.float32(1e30)
    prev = jnp.zeros((_TNQ, 1), jnp.float32)
    cols = []
    for t in range(_K):
        prev = jnp.min(jnp.where(key > prev, key, big), axis=1, keepdims=True)
        cols.append(prev)
    ids = jnp.bitwise_and(
        lax.bitcast_convert_type(jnp.concatenate(cols, axis=1), jnp.int32),
        jnp.int32(2047))
    o_ref[0] = jnp.transpose(ids) + pl.program_id(0) * _N


def _knn(q8, p8t):
    return pl.pallas_call(
        _knn_body,
        grid=(q8.shape[0], _N // _TNQ),
        in_specs=[
            pl.BlockSpec((1, _TNQ, 8), lambda b, i: (b, i, 0)),
            pl.BlockSpec((1, 8, _N), lambda b, i: (b, 0, 0)),
        ],
        out_specs=pl.BlockSpec((1, _K, _TNQ), lambda b, i: (b, 0, i)),
        out_shape=jax.ShapeDtypeStruct((q8.shape[0], _K, _N), jnp.int32),
    )(q8, p8t)


def _sc_gather(table, idx2):
    rows, n = idx2.shape
    num = rows * n

    @pl.kernel(
        out_type=jax.ShapeDtypeStruct((num, _CW), jnp.int32),
        mesh=plsc.VectorSubcoreMesh(core_axis_name="c", subcore_axis_name="s"),
    )
    def gather_kernel(t_hbm, i_hbm, o_hbm):
        def body(i_vmem, o_vmem):
            pltpu.sync_copy(t_hbm.at[i_vmem.at[0]], o_vmem)

        pltpu.emit_pipeline(
            body,
            grid=(rows, n // _GW),
            in_specs=[pl.BlockSpec((1, _GW), index_map=lambda r, c: (r, c))],
            out_specs=[pl.BlockSpec((_GW, _CW),
                                    index_map=lambda r, c: (r * (n // _GW) + c, 0))],
            core_axis_name=("c", "s"),
            dimension_semantics=(pltpu.PARALLEL, pltpu.PARALLEL),
        )(i_hbm, o_hbm)

    return gather_kernel(table, idx2)


def _mlp_body(g_ref, pt_ref,
              w1e_ref, w1o_ref, w1q_ref, b1_ref,
              w2_ref, b2_ref, w3_ref, b3_ref,
              wse_ref, wso_ref, wsq_ref, bs_ref,
              wo1_ref, bo1_ref, wo2_ref, bo2_ref, o_ref):
    bf16 = jnp.bfloat16
    f32 = jnp.float32
    g = g_ref[0]
    ge = lax.bitcast_convert_type(jnp.left_shift(g, 16), f32).astype(bf16)
    go = lax.bitcast_convert_type(
        jnp.bitwise_and(g, jnp.int32(-65536)), f32).astype(bf16)
    pt = pt_ref[0].astype(bf16)

    gef = ge.reshape(_K * _TN, _CW)
    gof = go.reshape(_K * _TN, _CW)

    h = (jnp.dot(gef, w1e_ref[...], preferred_element_type=f32)
         + jnp.dot(gof, w1o_ref[...], preferred_element_type=f32))
    h = h.reshape(_K, _TN, 128)
    h = h - jnp.dot(pt, w1q_ref[...], preferred_element_type=f32)[None, :, :]
    h = jnp.maximum(h + b1_ref[...], 0.0)
    h = h.reshape(_K * _TN, 128).astype(bf16)
    h = jnp.maximum(jnp.dot(h, w2_ref[...], preferred_element_type=f32)
                    + b2_ref[...], 0.0).astype(bf16)
    h = jnp.maximum(jnp.dot(h, w3_ref[...], preferred_element_type=f32)
                    + b3_ref[...], 0.0)
    h = h.astype(bf16)

    gem = ge[0]
    gom = go[0]
    for k in range(1, _K):
        gem = jnp.maximum(gem, ge[k])
        gom = jnp.maximum(gom, go[k])
    sk = (jnp.dot(gem, wse_ref[...], preferred_element_type=f32)
          + jnp.dot(gom, wso_ref[...], preferred_element_type=f32)
          - jnp.dot(pt, wsq_ref[...], preferred_element_type=f32))
    sk = jnp.maximum(sk + bs_ref[...], 0.0)

    acc = jnp.dot(h[0:_TN], wo1_ref[0], preferred_element_type=f32)
    for k in range(1, _K):
        acc = acc + jnp.dot(h[k * _TN:(k + 1) * _TN], wo1_ref[k],
                            preferred_element_type=f32)
    out1 = (jnp.maximum(acc + bo1_ref[...], 0.0) + sk).astype(bf16)
    out = jnp.maximum(jnp.dot(out1, wo2_ref[...], preferred_element_type=f32)
                      + bo2_ref[...], 0.0)
    o_ref[0] = out


def _mlp(g4, p8, weights):
    full = lambda shape: pl.BlockSpec(shape, lambda b, i: tuple(0 for _ in shape))
    w_specs = [
        full((_CW, 128)), full((_CW, 128)), full((8, 128)), full((1, 128)),
        full((128, 128)), full((1, 128)), full((128, 256)), full((1, 256)),
        full((_CW, 256)), full((_CW, 256)), full((8, 256)), full((1, 256)),
        full((_K, 256, 256)), full((1, 256)), full((256, 256)), full((1, 256)),
    ]
    return pl.pallas_call(
        _mlp_body,
        grid=(g4.shape[0], _N // _TN),
        in_specs=[
            pl.BlockSpec((1, _K, _TN, _CW), lambda b, i: (b, 0, i, 0)),
            pl.BlockSpec((1, _TN, 8), lambda b, i: (b, i, 0)),
        ] + w_specs,
        out_specs=pl.BlockSpec((1, _TN, 256), lambda b, i: (b, i, 0)),
        out_shape=jax.ShapeDtypeStruct((g4.shape[0], _N, 256), jnp.float32),
    )(g4, p8, *weights)


def kernel(points, point_features, query_points, W_skip, b_skip,
           W1, b1, W2, b2, W3, b3, Wo1, bo1, Wo2, bo2):
    f32 = jnp.float32
    pad5 = lambda x: jnp.pad(x, ((0, 0), (0, 0), (0, 5)))
    p8 = pad5(jnp.transpose(points, (0, 2, 1)))
    q8 = pad5(jnp.transpose(query_points, (0, 2, 1)))
    p8t = jnp.transpose(p8, (0, 2, 1))
    ft = jnp.transpose(point_features, (0, 2, 1))

    table = jnp.concatenate(
        [ft, p8[:, :, 0:3], jnp.zeros((_B, _N, _CP - _CIN - 3), f32)],
        axis=2).astype(jnp.bfloat16)
    table = lax.bitcast_convert_type(
        table.reshape(_B, _N, _CW, 2), jnp.int32)

    pad_w = lambda w: jnp.pad(w, ((0, 5), (0, 0)))
    W1t, W2t, W3t = W1.T, W2.T, W3.T
    Wst, Wo2t = W_skip.T, Wo2.T

    def tab_weights(Wt):
        out = Wt.shape[1]
        wtab = jnp.concatenate([
            Wt[3:3 + _CIN, :],
            Wt[0:3, :] + Wt[131:134, :],
            jnp.zeros((_CP - _CIN - 3, out), f32)], axis=0)
        return wtab[0::2, :], wtab[1::2, :]

    w1e, w1o = tab_weights(W1t)
    w1q = pad_w(W1t[131:134, :])
    wse, wso = tab_weights(Wst)
    wsq = pad_w(Wst[131:134, :])
    wo1t = jnp.transpose(Wo1, (1, 2, 0))
    row = lambda b: b.reshape(1, -1)
    bf = lambda w: w.astype(jnp.bfloat16)
    weights = [bf(w1e), bf(w1o), bf(w1q), row(b1), bf(W2t), row(b2),
               bf(W3t), row(b3), bf(wse), bf(wso), bf(wsq), row(b_skip),
               bf(wo1t), row(bo1), bf(Wo2t), row(bo2)]

    hs = []
    for b in range(_B):
        gidx_b = _knn(q8[b:b + 1], p8t[b:b + 1])
        g_b = _sc_gather(table[b], gidx_b.reshape(_K, _N))
        hs.append(_mlp(g_b.reshape(1, _K, _N, _CW), p8[b:b + 1], weights))
    h = jnp.concatenate(hs, axis=0)
    return (points, jnp.transpose(h, (0, 2, 1)))


# speedup vs baseline: 1.7383x; 1.0885x over previous
import jax
import jax.numpy as jnp
from jax import lax
from jax.experimental import pallas as pl
from jax.experimental.pallas import tpu as pltpu
from jax.experimental.pallas import tpu_sc as plsc

_B, _N, _K = 4, 2048, 16
_CIN = 128
_CP = 256
_CW = 128
_TNQ = 256
_TN = 256
_GW = 128


def _knn_body(q_ref, p_ref, o_ref):
    q = q_ref[0]
    p = p_ref[0]
    q2 = jnp.sum(q * q, axis=1, keepdims=True)
    p2 = jnp.sum(p * p, axis=0, keepdims=True)
    qp = jnp.dot(q, p, preferred_element_type=jnp.float32)
    d = q2 + p2 - 2.0 * qp
    iota = lax.broadcasted_iota(jnp.int32, d.shape, 1)
    bound = jnp.max(jnp.min(d.reshape(_TNQ, _K, _N // _K), axis=2),
                    axis=1, keepdims=True)
    scale = jnp.float32(2 ** 19 - 2) / jnp.maximum(bound, jnp.float32(1e-30))
    ki = jnp.clip((d * scale).astype(jnp.int32),
                  jnp.int32(0), jnp.int32(2 ** 19 - 1))
    key = lax.bitcast_convert_type(
        jnp.bitwise_or(jnp.left_shift(ki, 11), iota) + jnp.int32(2 ** 28),
        jnp.float32)
    big = jnp.float32(1e30)
    prev = jnp.zeros((_TNQ, 1), jnp.float32)
    cols = []
    for t in range(_K):
        prev = jnp.min(jnp.where(key > prev, key, big), axis=1, keepdims=True)
        cols.append(prev)
    ids = jnp.bitwise_and(
        lax.bitcast_convert_type(jnp.concatenate(cols, axis=1), jnp.int32),
        jnp.int32(2047))
    o_ref[0] = jnp.transpose(ids) + pl.program_id(0) * _N


def _knn(q8, p8t):
    return pl.pallas_call(
        _knn_body,
        grid=(q8.shape[0], _N // _TNQ),
        in_specs=[
            pl.BlockSpec((1, _TNQ, 8), lambda b, i: (b, i, 0)),
            pl.BlockSpec((1, 8, _N), lambda b, i: (b, 0, 0)),
        ],
        out_specs=pl.BlockSpec((1, _K, _TNQ), lambda b, i: (b, 0, i)),
        out_shape=jax.ShapeDtypeStruct((q8.shape[0], _K, _N), jnp.int32),
    )(q8, p8t)


def _pack_body(f_ref, p_ref, o_ref):
    i32 = jnp.int32

    def rb(x):
        b = lax.bitcast_convert_type(x, i32)
        return jnp.right_shift(
            b + i32(0x7FFF) + jnp.bitwise_and(jnp.right_shift(b, 16), 1), 16)

    ft = jnp.transpose(f_ref[0])
    pt = jnp.transpose(p_ref[0])
    lo = jnp.bitwise_and(rb(ft), i32(0xFFFF))
    hi = jnp.pad(jnp.left_shift(rb(pt), 16), ((0, 0), (0, _CW - 8)))
    o_ref[0] = jnp.bitwise_or(lo, hi)


def _pack(pf, p8t):
    return pl.pallas_call(
        _pack_body,
        grid=(pf.shape[0],),
        in_specs=[
            pl.BlockSpec((1, _CIN, _N), lambda b: (b, 0, 0)),
            pl.BlockSpec((1, 8, _N), lambda b: (b, 0, 0)),
        ],
        out_specs=pl.BlockSpec((1, _N, _CW), lambda b: (b, 0, 0)),
        out_shape=jax.ShapeDtypeStruct((pf.shape[0], _N, _CW), jnp.int32),
    )(pf, p8t)


def _sc_gather(table, idx2):
    rows, n = idx2.shape
    num = rows * n

    @pl.kernel(
        out_type=jax.ShapeDtypeStruct((num, _CW), jnp.int32),
        mesh=plsc.VectorSubcoreMesh(core_axis_name="c", subcore_axis_name="s"),
    )
    def gather_kernel(t_hbm, i_hbm, o_hbm):
        def body(i_vmem, o_vmem):
            pltpu.sync_copy(t_hbm.at[i_vmem.at[0]], o_vmem)

        pltpu.emit_pipeline(
            body,
            grid=(rows, n // _GW),
            in_specs=[pl.BlockSpec((1, _GW), index_map=lambda r, c: (r, c))],
            out_specs=[pl.BlockSpec((_GW, _CW),
                                    index_map=lambda r, c: (r * (n // _GW) + c, 0))],
            core_axis_name=("c", "s"),
            dimension_semantics=(pltpu.PARALLEL, pltpu.PARALLEL),
        )(i_hbm, o_hbm)

    return gather_kernel(table, idx2)


def _mlp_body(g_ref, pt_ref,
              w1e_ref, w1o_ref, w1q_ref, b1_ref,
              w2_ref, b2_ref, w3_ref, b3_ref,
              wse_ref, wso_ref, wsq_ref, bs_ref,
              wo1_ref, bo1_ref, wo2_ref, bo2_ref, o_ref):
    bf16 = jnp.bfloat16
    f32 = jnp.float32
    g = g_ref[0]
    ge = lax.bitcast_convert_type(jnp.left_shift(g, 16), f32).astype(bf16)
    go = lax.bitcast_convert_type(
        jnp.bitwise_and(g[:, :, 0:8], jnp.int32(-65536)), f32).astype(bf16)
    pt = pt_ref[0].astype(bf16)

    gef = ge.reshape(_K * _TN, _CW)
    gof = go.reshape(_K * _TN, 8)

    h = (jnp.dot(gef, w1e_ref[...], preferred_element_type=f32)
         + jnp.dot(gof, w1o_ref[...], preferred_element_type=f32))
    h = h.reshape(_K, _TN, 128)
    h = h - jnp.dot(pt, w1q_ref[...], preferred_element_type=f32)[None, :, :]
    h = jnp.maximum(h + b1_ref[...], 0.0)
    h = h.reshape(_K * _TN, 128).astype(bf16)
    h = jnp.maximum(jnp.dot(h, w2_ref[...], preferred_element_type=f32)
                    + b2_ref[...], 0.0).astype(bf16)
    h = jnp.maximum(jnp.dot(h, w3_ref[...], preferred_element_type=f32)
                    + b3_ref[...], 0.0)
    h = h.astype(bf16)

    gem = ge[0]
    gom = go[0]
    for k in range(1, _K):
        gem = jnp.maximum(gem, ge[k])
        gom = jnp.maximum(gom, go[k])
    sk = (jnp.dot(gem, wse_ref[...], preferred_element_type=f32)
          + jnp.dot(gom, wso_ref[...], preferred_element_type=f32)
          - jnp.dot(pt, wsq_ref[...], preferred_element_type=f32))
    sk = jnp.maximum(sk + bs_ref[...], 0.0)

    acc = jnp.dot(h[0:_TN], wo1_ref[0], preferred_element_type=f32)
    for k in range(1, _K):
        acc = acc + jnp.dot(h[k * _TN:(k + 1) * _TN], wo1_ref[k],
                            preferred_element_type=f32)
    out1 = (jnp.maximum(acc + bo1_ref[...], 0.0) + sk).astype(bf16)
    out = jnp.maximum(jnp.dot(out1, wo2_ref[...], preferred_element_type=f32)
                      + bo2_ref[...], 0.0)
    o_ref[0] = out


def _mlp(g4, p8, weights):
    full = lambda shape: pl.BlockSpec(shape, lambda b, i: tuple(0 for _ in shape))
    w_specs = [
        full((_CW, 128)), full((8, 128)), full((8, 128)), full((1, 128)),
        full((128, 128)), full((1, 128)), full((128, 256)), full((1, 256)),
        full((_CW, 256)), full((8, 256)), full((8, 256)), full((1, 256)),
        full((_K, 256, 256)), full((1, 256)), full((256, 256)), full((1, 256)),
    ]
    return pl.pallas_call(
        _mlp_body,
        grid=(g4.shape[0], _N // _TN),
        in_specs=[
            pl.BlockSpec((1, _K, _TN, _CW), lambda b, i: (b, 0, i, 0)),
            pl.BlockSpec((1, _TN, 8), lambda b, i: (b, i, 0)),
        ] + w_specs,
        out_specs=pl.BlockSpec((1, _TN, 256), lambda b, i: (b, i, 0)),
        out_shape=jax.ShapeDtypeStruct((g4.shape[0], _N, 256), jnp.float32),
    )(g4, p8, *weights)


def kernel(points, point_features, query_points, W_skip, b_skip,
           W1, b1, W2, b2, W3, b3, Wo1, bo1, Wo2, bo2):
    f32 = jnp.float32
    pad5 = lambda x: jnp.pad(x, ((0, 0), (0, 0), (0, 5)))
    p8 = pad5(jnp.transpose(points, (0, 2, 1)))
    q8 = pad5(jnp.transpose(query_points, (0, 2, 1)))
    p8t = jnp.transpose(p8, (0, 2, 1))

    table = _pack(point_features, p8t)

    pad_w = lambda w: jnp.pad(w, ((0, 5), (0, 0)))
    W1t, W2t, W3t = W1.T, W2.T, W3.T
    Wst, Wo2t = W_skip.T, Wo2.T

    w1e = W1t[3:3 + _CIN, :]
    w1o = pad_w(W1t[0:3, :] + W1t[131:134, :])
    w1q = pad_w(W1t[131:134, :])
    wse = Wst[3:3 + _CIN, :]
    wso = pad_w(Wst[0:3, :] + Wst[131:134, :])
    wsq = pad_w(Wst[131:134, :])
    wo1t = jnp.transpose(Wo1, (1, 2, 0))
    row = lambda b: b.reshape(1, -1)
    bf = lambda w: w.astype(jnp.bfloat16)
    weights = [bf(w1e), bf(w1o), bf(w1q), row(b1), bf(W2t), row(b2),
               bf(W3t), row(b3), bf(wse), bf(wso), bf(wsq), row(b_skip),
               bf(wo1t), row(bo1), bf(Wo2t), row(bo2)]

    hs = []
    for b in range(_B):
        gidx_b = _knn(q8[b:b + 1], p8t[b:b + 1])
        g_b = _sc_gather(table[b], gidx_b.reshape(_K, _N))
        hs.append(_mlp(g_b.reshape(1, _K, _N, _CW), p8[b:b + 1], weights))
    h = jnp.concatenate(hs, axis=0)
    return (points, jnp.transpose(h, (0, 2, 1)))


# speedup vs baseline: 1.7595x; 1.0122x over previous
import jax
import jax.numpy as jnp
from jax import lax
from jax.experimental import pallas as pl
from jax.experimental.pallas import tpu as pltpu
from jax.experimental.pallas import tpu_sc as plsc

_B, _N, _K = 4, 2048, 16
_CIN = 128
_CP = 256
_CW = 128
_TNQ = 512
_TN = 256
_GW = 128


def _knn_body(q_ref, p_ref, o_ref):
    q = q_ref[0]
    p = p_ref[0]
    q2 = jnp.sum(q * q, axis=1, keepdims=True)
    p2 = jnp.sum(p * p, axis=0, keepdims=True)
    qp = jnp.dot(q, p, preferred_element_type=jnp.float32)
    d = q2 + p2 - 2.0 * qp
    iota = lax.broadcasted_iota(jnp.int32, d.shape, 1)
    bound = jnp.max(jnp.min(d.reshape(_TNQ, _K, _N // _K), axis=2),
                    axis=1, keepdims=True)
    scale = jnp.float32(2 ** 19 - 2) / jnp.maximum(bound, jnp.float32(1e-30))
    ki = jnp.clip((d * scale).astype(jnp.int32),
                  jnp.int32(0), jnp.int32(2 ** 19 - 1))
    key = lax.bitcast_convert_type(
        jnp.bitwise_or(jnp.left_shift(ki, 11), iota) + jnp.int32(2 ** 28),
        jnp.float32)
    big = jnp.float32(1e30)
    prev = jnp.zeros((_TNQ, 1), jnp.float32)
    cols = []
    for t in range(_K):
        prev = jnp.min(jnp.where(key > prev, key, big), axis=1, keepdims=True)
        cols.append(prev)
    ids = jnp.bitwise_and(
        lax.bitcast_convert_type(jnp.concatenate(cols, axis=1), jnp.int32),
        jnp.int32(2047))
    o_ref[0] = jnp.transpose(ids) + pl.program_id(0) * _N


def _knn(q8, p8t):
    return pl.pallas_call(
        _knn_body,
        grid=(q8.shape[0], _N // _TNQ),
        in_specs=[
            pl.BlockSpec((1, _TNQ, 8), lambda b, i: (b, i, 0)),
            pl.BlockSpec((1, 8, _N), lambda b, i: (b, 0, 0)),
        ],
        out_specs=pl.BlockSpec((1, _K, _TNQ), lambda b, i: (b, 0, i)),
        out_shape=jax.ShapeDtypeStruct((q8.shape[0], _K, _N), jnp.int32),
    )(q8, p8t)


def _pack_body(f_ref, p_ref, o_ref):
    i32 = jnp.int32

    def rb(x):
        b = lax.bitcast_convert_type(x, i32)
        return jnp.right_shift(
            b + i32(0x7FFF) + jnp.bitwise_and(jnp.right_shift(b, 16), 1), 16)

    ft = jnp.transpose(f_ref[0])
    pt = jnp.transpose(p_ref[0])
    lo = jnp.bitwise_and(rb(ft), i32(0xFFFF))
    hi = jnp.pad(jnp.left_shift(rb(pt), 16), ((0, 0), (0, _CW - 8)))
    o_ref[0] = jnp.bitwise_or(lo, hi)


def _pack(pf, p8t):
    return pl.pallas_call(
        _pack_body,
        grid=(pf.shape[0],),
        in_specs=[
            pl.BlockSpec((1, _CIN, _N), lambda b: (b, 0, 0)),
            pl.BlockSpec((1, 8, _N), lambda b: (b, 0, 0)),
        ],
        out_specs=pl.BlockSpec((1, _N, _CW), lambda b: (b, 0, 0)),
        out_shape=jax.ShapeDtypeStruct((pf.shape[0], _N, _CW), jnp.int32),
    )(pf, p8t)


def _sc_gather(table, idx2):
    rows, n = idx2.shape
    num = rows * n

    @pl.kernel(
        out_type=jax.ShapeDtypeStruct((num, _CW), jnp.int32),
        mesh=plsc.VectorSubcoreMesh(core_axis_name="c", subcore_axis_name="s"),
    )
    def gather_kernel(t_hbm, i_hbm, o_hbm):
        def body(i_vmem, o_vmem):
            pltpu.sync_copy(t_hbm.at[i_vmem.at[0]], o_vmem)

        pltpu.emit_pipeline(
            body,
            grid=(rows, n // _GW),
            in_specs=[pl.BlockSpec((1, _GW), index_map=lambda r, c: (r, c))],
            out_specs=[pl.BlockSpec((_GW, _CW),
                                    index_map=lambda r, c: (r * (n // _GW) + c, 0))],
            core_axis_name=("c", "s"),
            dimension_semantics=(pltpu.PARALLEL, pltpu.PARALLEL),
        )(i_hbm, o_hbm)

    return gather_kernel(table, idx2)


def _mlp_body(g_ref, pt_ref,
              w1e_ref, w1o_ref, w1q_ref, b1_ref,
              w2_ref, b2_ref, w3_ref, b3_ref,
              wse_ref, wso_ref, wsq_ref, bs_ref,
              wo1_ref, bo1_ref, wo2_ref, bo2_ref, o_ref):
    bf16 = jnp.bfloat16
    f32 = jnp.float32
    g = g_ref[0]
    ge = lax.bitcast_convert_type(jnp.left_shift(g, 16), f32).astype(bf16)
    go = lax.bitcast_convert_type(
        jnp.bitwise_and(g[:, :, 0:8], jnp.int32(-65536)), f32).astype(bf16)
    pt = pt_ref[0].astype(bf16)

    gef = ge.reshape(_K * _TN, _CW)
    gof = go.reshape(_K * _TN, 8)

    h = (jnp.dot(gef, w1e_ref[...], preferred_element_type=f32)
         + jnp.dot(gof, w1o_ref[...], preferred_element_type=f32))
    h = h.reshape(_K, _TN, 128)
    h = h - jnp.dot(pt, w1q_ref[...], preferred_element_type=f32)[None, :, :]
    h = jnp.maximum(h + b1_ref[...], 0.0)
    h = h.reshape(_K * _TN, 128).astype(bf16)
    h = jnp.maximum(jnp.dot(h, w2_ref[...], preferred_element_type=f32)
                    + b2_ref[...], 0.0).astype(bf16)
    h = jnp.maximum(jnp.dot(h, w3_ref[...], preferred_element_type=f32)
                    + b3_ref[...], 0.0)
    h = h.astype(bf16)

    gem = ge[0]
    gom = go[0]
    for k in range(1, _K):
        gem = jnp.maximum(gem, ge[k])
        gom = jnp.maximum(gom, go[k])
    sk = (jnp.dot(gem, wse_ref[...], preferred_element_type=f32)
          + jnp.dot(gom, wso_ref[...], preferred_element_type=f32)
          - jnp.dot(pt, wsq_ref[...], preferred_element_type=f32))
    sk = jnp.maximum(sk + bs_ref[...], 0.0)

    acc = jnp.dot(h[0:_TN], wo1_ref[0], preferred_element_type=f32)
    for k in range(1, _K):
        acc = acc + jnp.dot(h[k * _TN:(k + 1) * _TN], wo1_ref[k],
                            preferred_element_type=f32)
    out1 = (jnp.maximum(acc + bo1_ref[...], 0.0) + sk).astype(bf16)
    out = jnp.maximum(jnp.dot(out1, wo2_ref[...], preferred_element_type=f32)
                      + bo2_ref[...], 0.0)
    o_ref[0] = out


def _mlp(g4, p8, weights):
    full = lambda shape: pl.BlockSpec(shape, lambda b, i: tuple(0 for _ in shape))
    w_specs = [
        full((_CW, 128)), full((8, 128)), full((8, 128)), full((1, 128)),
        full((128, 128)), full((1, 128)), full((128, 256)), full((1, 256)),
        full((_CW, 256)), full((8, 256)), full((8, 256)), full((1, 256)),
        full((_K, 256, 256)), full((1, 256)), full((256, 256)), full((1, 256)),
    ]
    return pl.pallas_call(
        _mlp_body,
        grid=(g4.shape[0], _N // _TN),
        in_specs=[
            pl.BlockSpec((1, _K, _TN, _CW), lambda b, i: (b, 0, i, 0)),
            pl.BlockSpec((1, _TN, 8), lambda b, i: (b, i, 0)),
        ] + w_specs,
        out_specs=pl.BlockSpec((1, _TN, 256), lambda b, i: (b, i, 0)),
        out_shape=jax.ShapeDtypeStruct((g4.shape[0], _N, 256), jnp.float32),
    )(g4, p8, *weights)


def kernel(points, point_features, query_points, W_skip, b_skip,
           W1, b1, W2, b2, W3, b3, Wo1, bo1, Wo2, bo2):
    f32 = jnp.float32
    pad5 = lambda x: jnp.pad(x, ((0, 0), (0, 0), (0, 5)))
    p8 = pad5(jnp.transpose(points, (0, 2, 1)))
    q8 = pad5(jnp.transpose(query_points, (0, 2, 1)))
    p8t = jnp.transpose(p8, (0, 2, 1))

    table = _pack(point_features, p8t)

    pad_w = lambda w: jnp.pad(w, ((0, 5), (0, 0)))
    W1t, W2t, W3t = W1.T, W2.T, W3.T
    Wst, Wo2t = W_skip.T, Wo2.T

    w1e = W1t[3:3 + _CIN, :]
    w1o = pad_w(W1t[0:3, :] + W1t[131:134, :])
    w1q = pad_w(W1t[131:134, :])
    wse = Wst[3:3 + _CIN, :]
    wso = pad_w(Wst[0:3, :] + Wst[131:134, :])
    wsq = pad_w(Wst[131:134, :])
    wo1t = jnp.transpose(Wo1, (1, 2, 0))
    row = lambda b: b.reshape(1, -1)
    bf = lambda w: w.astype(jnp.bfloat16)
    weights = [bf(w1e), bf(w1o), bf(w1q), row(b1), bf(W2t), row(b2),
               bf(W3t), row(b3), bf(wse), bf(wso), bf(wsq), row(b_skip),
               bf(wo1t), row(bo1), bf(Wo2t), row(bo2)]

    hs = []
    for b in range(_B):
        gidx_b = _knn(q8[b:b + 1], p8t[b:b + 1])
        g_b = _sc_gather(table[b], gidx_b.reshape(_K, _N))
        hs.append(_mlp(g_b.reshape(1, _K, _N, _CW), p8[b:b + 1], weights))
    h = jnp.concatenate(hs, axis=0)
    return (points, jnp.transpose(h, (0, 2, 1)))


# speedup vs baseline: 1.9144x; 1.0880x over previous
import jax
import jax.numpy as jnp
from jax import lax
from jax.experimental import pallas as pl
from jax.experimental.pallas import tpu as pltpu
from jax.experimental.pallas import tpu_sc as plsc

_B, _N, _K = 4, 2048, 16
_CIN = 128
_CP = 256
_CW = 128
_TNQ = 512
_TN = 256
_GW = 128


def _knn_body(q_ref, p_ref, o_ref):
    q = jnp.transpose(q_ref[0])
    p = p_ref[0]
    q2 = jnp.sum(q * q, axis=1, keepdims=True)
    p2 = jnp.sum(p * p, axis=0, keepdims=True)
    qp = jnp.dot(q, p, preferred_element_type=jnp.float32)
    d = q2 + p2 - 2.0 * qp
    iota = lax.broadcasted_iota(jnp.int32, d.shape, 1)
    bound = jnp.max(jnp.min(d.reshape(_TNQ, _K, _N // _K), axis=2),
                    axis=1, keepdims=True)
    scale = jnp.float32(2 ** 19 - 2) / jnp.maximum(bound, jnp.float32(1e-30))
    ki = jnp.clip((d * scale).astype(jnp.int32),
                  jnp.int32(0), jnp.int32(2 ** 19 - 1))
    key = lax.bitcast_convert_type(
        jnp.bitwise_or(jnp.left_shift(ki, 11), iota) + jnp.int32(2 ** 28),
        jnp.float32)
    big = jnp.float32(1e30)
    prev = jnp.zeros((_TNQ, 1), jnp.float32)
    cols = []
    for t in range(_K):
        prev = jnp.min(jnp.where(key > prev, key, big), axis=1, keepdims=True)
        cols.append(prev)
    ids = jnp.bitwise_and(
        lax.bitcast_convert_type(jnp.concatenate(cols, axis=1), jnp.int32),
        jnp.int32(2047))
    o_ref[0] = jnp.transpose(ids) + pl.program_id(0) * _N


def _knn(q8t, p8t):
    return pl.pallas_call(
        _knn_body,
        grid=(q8t.shape[0], _N // _TNQ),
        in_specs=[
            pl.BlockSpec((1, 8, _TNQ), lambda b, i: (b, 0, i)),
            pl.BlockSpec((1, 8, _N), lambda b, i: (b, 0, 0)),
        ],
        out_specs=pl.BlockSpec((1, _K, _TNQ), lambda b, i: (b, 0, i)),
        out_shape=jax.ShapeDtypeStruct((q8t.shape[0], _K, _N), jnp.int32),
    )(q8t, p8t)


def _pack_body(f_ref, p_ref, o_ref):
    i32 = jnp.int32

    def rb(x):
        b = lax.bitcast_convert_type(x, i32)
        return jnp.right_shift(
            b + i32(0x7FFF) + jnp.bitwise_and(jnp.right_shift(b, 16), 1), 16)

    ft = jnp.transpose(f_ref[0])
    pt = jnp.transpose(p_ref[0])
    lo = jnp.bitwise_and(rb(ft), i32(0xFFFF))
    hi = jnp.pad(jnp.left_shift(rb(pt), 16), ((0, 0), (0, _CW - 8)))
    o_ref[0] = jnp.bitwise_or(lo, hi)


def _pack(pf, p8t):
    return pl.pallas_call(
        _pack_body,
        grid=(pf.shape[0],),
        in_specs=[
            pl.BlockSpec((1, _CIN, _N), lambda b: (b, 0, 0)),
            pl.BlockSpec((1, 8, _N), lambda b: (b, 0, 0)),
        ],
        out_specs=pl.BlockSpec((1, _N, _CW), lambda b: (b, 0, 0)),
        out_shape=jax.ShapeDtypeStruct((pf.shape[0], _N, _CW), jnp.int32),
    )(pf, p8t)


def _sc_gather(table, idx2):
    rows, n = idx2.shape
    num = rows * n

    @pl.kernel(
        out_type=jax.ShapeDtypeStruct((num, _CW), jnp.int32),
        mesh=plsc.VectorSubcoreMesh(core_axis_name="c", subcore_axis_name="s"),
    )
    def gather_kernel(t_hbm, i_hbm, o_hbm):
        def body(i_vmem, o_vmem):
            pltpu.sync_copy(t_hbm.at[i_vmem.at[0]], o_vmem)

        pltpu.emit_pipeline(
            body,
            grid=(rows, n // _GW),
            in_specs=[pl.BlockSpec((1, _GW), index_map=lambda r, c: (r, c))],
            out_specs=[pl.BlockSpec((_GW, _CW),
                                    index_map=lambda r, c: (r * (n // _GW) + c, 0))],
            core_axis_name=("c", "s"),
            dimension_semantics=(pltpu.PARALLEL, pltpu.PARALLEL),
        )(i_hbm, o_hbm)

    return gather_kernel(table, idx2)


def _mlp_body(g_ref, pt_ref,
              w1e_ref, w1o_ref, w1q_ref, b1_ref,
              w2_ref, b2_ref, w3_ref, b3_ref,
              wse_ref, wso_ref, wsq_ref, bs_ref,
              wo1_ref, bo1_ref, wo2_ref, bo2_ref, o_ref):
    bf16 = jnp.bfloat16
    f32 = jnp.float32
    g = g_ref[0]
    ge = lax.bitcast_convert_type(jnp.left_shift(g, 16), f32).astype(bf16)
    go = lax.bitcast_convert_type(
        jnp.bitwise_and(g[:, :, 0:8], jnp.int32(-65536)), f32).astype(bf16)
    pt = jnp.transpose(pt_ref[0]).astype(bf16)

    gef = ge.reshape(_K * _TN, _CW)
    gof = go.reshape(_K * _TN, 8)

    h = (jnp.dot(gef, w1e_ref[...], preferred_element_type=f32)
         + jnp.dot(gof, w1o_ref[...], preferred_element_type=f32))
    h = h.reshape(_K, _TN, 128)
    h = h - jnp.dot(pt, w1q_ref[...], preferred_element_type=f32)[None, :, :]
    h = jnp.maximum(h + b1_ref[...], 0.0)
    h = h.reshape(_K * _TN, 128).astype(bf16)
    h = jnp.maximum(jnp.dot(h, w2_ref[...], preferred_element_type=f32)
                    + b2_ref[...], 0.0).astype(bf16)
    h = jnp.maximum(jnp.dot(h, w3_ref[...], preferred_element_type=f32)
                    + b3_ref[...], 0.0)
    h = h.astype(bf16)

    gem = ge[0]
    gom = go[0]
    for k in range(1, _K):
        gem = jnp.maximum(gem, ge[k])
        gom = jnp.maximum(gom, go[k])
    sk = (jnp.dot(gem, wse_ref[...], preferred_element_type=f32)
          + jnp.dot(gom, wso_ref[...], preferred_element_type=f32)
          - jnp.dot(pt, wsq_ref[...], preferred_element_type=f32))
    sk = jnp.maximum(sk + bs_ref[...], 0.0)

    acc = jnp.dot(h[0:_TN], wo1_ref[0], preferred_element_type=f32)
    for k in range(1, _K):
        acc = acc + jnp.dot(h[k * _TN:(k + 1) * _TN], wo1_ref[k],
                            preferred_element_type=f32)
    out1 = (jnp.maximum(acc + bo1_ref[...], 0.0) + sk).astype(bf16)
    out = jnp.maximum(jnp.dot(out1, wo2_ref[...], preferred_element_type=f32)
                      + bo2_ref[...], 0.0)
    o_ref[0] = out


def _mlp(g4, p8, weights):
    full = lambda shape: pl.BlockSpec(shape, lambda b, i: tuple(0 for _ in shape))
    w_specs = [
        full((_CW, 128)), full((8, 128)), full((8, 128)), full((1, 128)),
        full((128, 128)), full((1, 128)), full((128, 256)), full((1, 256)),
        full((_CW, 256)), full((8, 256)), full((8, 256)), full((1, 256)),
        full((_K, 256, 256)), full((1, 256)), full((256, 256)), full((1, 256)),
    ]
    return pl.pallas_call(
        _mlp_body,
        grid=(g4.shape[0], _N // _TN),
        in_specs=[
            pl.BlockSpec((1, _K, _TN, _CW), lambda b, i: (b, 0, i, 0)),
            pl.BlockSpec((1, 8, _TN), lambda b, i: (b, 0, i)),
        ] + w_specs,
        out_specs=pl.BlockSpec((1, _TN, 256), lambda b, i: (b, i, 0)),
        out_shape=jax.ShapeDtypeStruct((g4.shape[0], _N, 256), jnp.float32),
    )(g4, p8, *weights)


def kernel(points, point_features, query_points, W_skip, b_skip,
           W1, b1, W2, b2, W3, b3, Wo1, bo1, Wo2, bo2):
    p8t = jnp.pad(points, ((0, 0), (0, 5), (0, 0)))
    q8t = jnp.pad(query_points, ((0, 0), (0, 5), (0, 0)))

    table = _pack(point_features, p8t)

    pad_w = lambda w: jnp.pad(w, ((0, 5), (0, 0)))
    W1t, W2t, W3t = W1.T, W2.T, W3.T
    Wst, Wo2t = W_skip.T, Wo2.T

    w1e = W1t[3:3 + _CIN, :]
    w1o = pad_w(W1t[0:3, :] + W1t[131:134, :])
    w1q = pad_w(W1t[131:134, :])
    wse = Wst[3:3 + _CIN, :]
    wso = pad_w(Wst[0:3, :] + Wst[131:134, :])
    wsq = pad_w(Wst[131:134, :])
    wo1t = jnp.transpose(Wo1, (1, 2, 0))
    row = lambda b: b.reshape(1, -1)
    bf = lambda w: w.astype(jnp.bfloat16)
    weights = [bf(w1e), bf(w1o), bf(w1q), row(b1), bf(W2t), row(b2),
               bf(W3t), row(b3), bf(wse), bf(wso), bf(wsq), row(b_skip),
               bf(wo1t), row(bo1), bf(Wo2t), row(bo2)]

    hs = []
    for b in range(_B):
        gidx_b = _knn(q8t[b:b + 1], p8t[b:b + 1])
        g_b = _sc_gather(table[b], gidx_b.reshape(_K, _N))
        hs.append(_mlp(g_b.reshape(1, _K, _N, _CW), p8t[b:b + 1], weights))
    h = jnp.concatenate(hs, axis=0)
    return (points, jnp.transpose(h, (0, 2, 1)))
